# Initial kernel scaffold; baseline (speedup 1.0000x reference)
import jax
import jax.numpy as jnp
from jax.experimental import pallas as pl


def kernel(node_ids, edge_index, emb, W1, b1, g1, be1, W2, b2, g2, be2):
    raise NotImplementedError("write your pallas kernel here")


# speedup vs baseline: 9.7955x; 9.7955x over previous
import functools

import jax
import jax.numpy as jnp
from jax import lax
from jax.experimental import pallas as pl
from jax.experimental.pallas import tpu as pltpu
from jax.experimental.pallas import tpu_sc as plsc

N = 10000
P = 256
D = 128
E = 320000

NC = 2
NS = 16
NW = NC * NS

CHUNK = 128
EPW = 10112
E_PAD = NW * EPW
CHUNKS_PER_W = EPW // CHUNK

N_ACC = 10112
ROWS_PER_TILE = N_ACC // NS
N_DEG = 10240
DEG_PER_TILE = N_DEG // NS

_MESH = plsc.VectorSubcoreMesh(core_axis_name="c", subcore_axis_name="s")


def _sc_deg_body(dst_hbm, out_hbm, idx_v, ones_v, zero_v, acc_sh):
    c = lax.axis_index("c")
    s = lax.axis_index("s")
    one16 = jnp.full((16,), 1.0, jnp.float32)
    zero16 = jnp.zeros((16,), jnp.float32)
    for i in range(CHUNK // 16):
        ones_v[pl.ds(i * 16, 16)] = one16
    for i in range(DEG_PER_TILE // 16):
        zero_v[pl.ds(i * 16, 16)] = zero16
    pltpu.sync_copy(zero_v, acc_sh.at[pl.ds(s * DEG_PER_TILE, DEG_PER_TILE)])
    plsc.subcore_barrier()
    base = (s * NC + c) * EPW

    def body(j, carry):
        pltpu.sync_copy(dst_hbm.at[pl.ds(base + j * CHUNK, CHUNK)], idx_v)
        pltpu.sync_copy(ones_v, acc_sh.at[idx_v], add=True)
        return carry

    lax.fori_loop(0, CHUNKS_PER_W, body, 0)
    plsc.subcore_barrier()
    pltpu.sync_copy(acc_sh.at[pl.ds(s * DEG_PER_TILE, DEG_PER_TILE)],
                    out_hbm.at[c, pl.ds(s * DEG_PER_TILE, DEG_PER_TILE)])


_deg_kernel = pl.kernel(
    _sc_deg_body,
    out_type=jax.ShapeDtypeStruct((NC, N_DEG), jnp.float32),
    mesh=_MESH,
    scratch_types=[
        pltpu.VMEM((CHUNK,), jnp.int32),
        pltpu.VMEM((CHUNK,), jnp.float32),
        pltpu.VMEM((DEG_PER_TILE,), jnp.float32),
        pltpu.VMEM_SHARED((N_DEG,), jnp.float32),
    ],
)


def _sc_agg_body(src_hbm, dst_hbm, y_hbm, out_hbm, sidx_v, didx_v, rows_v, acc_sh):
    c = lax.axis_index("c")
    s = lax.axis_index("s")
    zero16 = jnp.zeros((16,), jnp.float32)

    def zbody(i, carry):
        r = i // (D // 16)
        l = i % (D // 16)
        rows_v[r, pl.ds(l * 16, 16)] = zero16
        return carry

    lax.fori_loop(0, CHUNK * (D // 16), zbody, 0)
    row0 = s * ROWS_PER_TILE
    for k in range(ROWS_PER_TILE // CHUNK):
        pltpu.sync_copy(rows_v, acc_sh.at[pl.ds(row0 + k * CHUNK, CHUNK)])
    rem = ROWS_PER_TILE % CHUNK
    pltpu.sync_copy(rows_v.at[pl.ds(0, rem)],
                    acc_sh.at[pl.ds(row0 + (ROWS_PER_TILE // CHUNK) * CHUNK, rem)])
    plsc.subcore_barrier()
    base = (s * NC + c) * EPW

    def body(j, carry):
        b = base + j * CHUNK
        pltpu.sync_copy(src_hbm.at[pl.ds(b, CHUNK)], sidx_v)
        pltpu.sync_copy(dst_hbm.at[pl.ds(b, CHUNK)], didx_v)
        pltpu.sync_copy(y_hbm.at[sidx_v], rows_v)
        pltpu.sync_copy(rows_v, acc_sh.at[didx_v], add=True)
        return carry

    lax.fori_loop(0, CHUNKS_PER_W, body, 0)
    plsc.subcore_barrier()
    pltpu.sync_copy(acc_sh.at[pl.ds(row0, ROWS_PER_TILE)],
                    out_hbm.at[c, pl.ds(row0, ROWS_PER_TILE)])


_agg_kernel = pl.kernel(
    _sc_agg_body,
    out_type=jax.ShapeDtypeStruct((NC, N_ACC, D), jnp.float32),
    mesh=_MESH,
    scratch_types=[
        pltpu.VMEM((CHUNK,), jnp.int32),
        pltpu.VMEM((CHUNK,), jnp.int32),
        pltpu.VMEM((CHUNK, D), jnp.float32),
        pltpu.VMEM_SHARED((N_ACC, D), jnp.float32),
    ],
)


BL = 2000
GRID = N // BL


def _tc1_body(degp_ref, nid_ref, emb_ref, w1_ref, dis_ref, x_ref, y1_ref):
    deg = degp_ref[0] + degp_ref[1] + 1.0
    dis = lax.rsqrt(deg)
    dis_ref[...] = dis
    nid = nid_ref[...]
    iota = lax.broadcasted_iota(jnp.int32, (BL, P), 1)
    onehot = jnp.where(nid == iota, 1.0, 0.0).astype(jnp.float32)
    x = jnp.dot(onehot, emb_ref[...], preferred_element_type=jnp.float32,
                precision=lax.Precision.HIGHEST)
    x_ref[...] = x
    h1 = jnp.dot(x, w1_ref[...], preferred_element_type=jnp.float32,
                 precision=lax.Precision.HIGHEST)
    y1_ref[...] = dis * h1


def _layer_norm(t, g, b):
    mu = jnp.mean(t, axis=1, keepdims=True)
    var = jnp.mean((t - mu) ** 2, axis=1, keepdims=True)
    return (t - mu) * lax.rsqrt(var + 1e-5) * g + b


def _tc2_body(p_ref, y1_ref, x_ref, dis_ref, b1_ref, g1_ref, be1_ref, w2_ref,
              x1_ref, y2_ref):
    agg = p_ref[0] + p_ref[1] + y1_ref[...]
    dis = dis_ref[...]
    out1 = dis * agg + b1_ref[...]
    x1 = _layer_norm(x_ref[...] + out1, g1_ref[...], be1_ref[...])
    x1_ref[...] = x1
    h2 = jnp.dot(x1, w2_ref[...], preferred_element_type=jnp.float32,
                 precision=lax.Precision.HIGHEST)
    y2_ref[...] = dis * h2


def _tc3_body(p_ref, y2_ref, x1_ref, dis_ref, b2_ref, g2_ref, be2_ref, out_ref):
    agg = p_ref[0] + p_ref[1] + y2_ref[...]
    out2 = dis_ref[...] * agg + b2_ref[...]
    out_ref[...] = _layer_norm(x1_ref[...] + out2, g2_ref[...], be2_ref[...])


def kernel(node_ids, edge_index, emb, W1, b1, g1, be1, W2, b2, g2, be2):
    src = edge_index[0]
    dst = edge_index[1]
    pad = E_PAD - E
    src_p = jnp.concatenate([src, jnp.zeros((pad,), src.dtype)]).astype(jnp.int32)
    dst_p = jnp.concatenate([dst, jnp.full((pad,), N, dst.dtype)]).astype(jnp.int32)

    deg_partial = _deg_kernel(dst_p)

    degp = deg_partial[:, :N].reshape(NC, N, 1)
    nid_col = node_ids.astype(jnp.int32).reshape(N, 1)

    row_spec = pl.BlockSpec((BL, D), lambda i: (i, 0))
    col_spec = pl.BlockSpec((BL, 1), lambda i: (i, 0))
    vec_spec = pl.BlockSpec((1, D), lambda i: (0, 0))
    w_spec = pl.BlockSpec((D, D), lambda i: (0, 0))
    p_spec = pl.BlockSpec((NC, BL, D), lambda i: (0, i, 0))

    dis, x, y1 = pl.pallas_call(
        _tc1_body,
        grid=(GRID,),
        in_specs=[
            pl.BlockSpec((NC, BL, 1), lambda i: (0, i, 0)),
            col_spec,
            pl.BlockSpec((P, D), lambda i: (0, 0)),
            w_spec,
        ],
        out_specs=(col_spec, row_spec, row_spec),
        out_shape=(
            jax.ShapeDtypeStruct((N, 1), jnp.float32),
            jax.ShapeDtypeStruct((N, D), jnp.float32),
            jax.ShapeDtypeStruct((N, D), jnp.float32),
        ),
    )(degp, nid_col, emb, W1)

    p1 = _agg_kernel(src_p, dst_p, y1)

    x1, y2 = pl.pallas_call(
        _tc2_body,
        grid=(GRID,),
        in_specs=[p_spec, row_spec, row_spec, col_spec,
                  vec_spec, vec_spec, vec_spec, w_spec],
        out_specs=(row_spec, row_spec),
        out_shape=(
            jax.ShapeDtypeStruct((N, D), jnp.float32),
            jax.ShapeDtypeStruct((N, D), jnp.float32),
        ),
    )(p1[:, :N], y1, x, dis, b1.reshape(1, D), g1.reshape(1, D),
      be1.reshape(1, D), W2)

    p2 = _agg_kernel(src_p, dst_p, y2)

    out = pl.pallas_call(
        _tc3_body,
        grid=(GRID,),
        in_specs=[p_spec, row_spec, row_spec, col_spec,
                  vec_spec, vec_spec, vec_spec],
        out_specs=row_spec,
        out_shape=jax.ShapeDtypeStruct((N, D), jnp.float32),
    )(p2[:, :N], y2, x1, dis, b2.reshape(1, D), g2.reshape(1, D),
      be2.reshape(1, D))

    return out
